# baseline (speedup 1.0000x reference)
import functools

import jax
import jax.numpy as jnp
from jax import lax
from jax.experimental import pallas as pl
from jax.experimental.pallas import tpu as pltpu
from jax.experimental.pallas import tpu_sc as plsc

N = 10000
E = 160000
D = 256
L = 5

NW = 32
NB = 320
NP = NW * NB
KC = 128
EP = E + 2 * KC
ET = E // NW
KH = 40
KE = 64
BM = 1024

_SC_PARAMS = pltpu.CompilerParams(
    use_tc_tiling_on_sc=False, needs_layout_passes=False
)
_MESH = plsc.VectorSubcoreMesh(core_axis_name="c", subcore_axis_name="s")

_f32 = jnp.float32
_i32 = jnp.int32


@functools.partial(
    pl.kernel,
    mesh=_MESH,
    compiler_params=_SC_PARAMS,
    out_type=(
        jax.ShapeDtypeStruct((NP, D), _f32),
        jax.ShapeDtypeStruct((2, NP), _f32),
    ),
    scratch_types=[
        pltpu.VMEM((KE,), _i32),
        pltpu.VMEM((KE, D), _f32),
        pltpu.VMEM((KE, D), _f32),
        pltpu.VMEM((KH,), _i32),
        pltpu.VMEM((48,), _f32),
        pltpu.VMEM((NP // 16,), _f32),
        pltpu.VMEM_SHARED((NP,), _f32),
        pltpu.SemaphoreType.DMA,
    ],
)
def _k0a(xi, ndi, ntab, dtab, rowr, h0, hist, idx_v, t1_v, t2_v, hidx_v,
         ones_v, zb_v, hist_sp, sem):
    c = lax.axis_index("c")
    s = lax.axis_index("s")
    w = c * 16 + s
    base = w * NB
    zslice = NP // 16

    zf = jnp.zeros((16,), _f32)
    of = zf + 1.0

    def _zb(i, _):
        zb_v[pl.ds(i * 16, 16)] = zf
        return 0

    lax.fori_loop(0, zslice // 16, _zb, 0)
    for k in range(3):
        ones_v[pl.ds(k * 16, 16)] = of

    pltpu.sync_copy(zb_v, hist_sp.at[pl.ds(pl.multiple_of(s * zslice, 8), zslice)])
    plsc.subcore_barrier()

    ebase = w * ET

    def _hchunk(ci, _):
        e0 = pl.multiple_of(ebase + ci * KH, 8)
        pltpu.sync_copy(rowr.at[pl.ds(e0, KH)], hidx_v)
        pltpu.sync_copy(ones_v.at[pl.ds(0, KH)], hist_sp.at[hidx_v],
                        add=True)
        return 0

    lax.fori_loop(0, ET // KH, _hchunk, 0)

    for oc in range(NB // KE):
        o = pl.multiple_of(base + oc * KE, 8)
        pltpu.sync_copy(xi.at[pl.ds(o, KE)], idx_v)
        pltpu.async_copy(ntab.at[idx_v], t1_v, sem).wait()
        pltpu.sync_copy(ndi.at[pl.ds(o, KE)], idx_v)
        pltpu.async_copy(dtab.at[idx_v], t2_v, sem).wait()

        def _addrow(r, _):
            for k in range(D // 16):
                t1_v[r, pl.ds(k * 16, 16)] = (
                    t1_v[r, pl.ds(k * 16, 16)] + t2_v[r, pl.ds(k * 16, 16)]
                )
            return 0

        lax.fori_loop(0, KE, _addrow, 0)
        pltpu.sync_copy(t1_v, h0.at[pl.ds(o, KE)])

    plsc.subcore_barrier()

    @pl.when(s == 0)
    def _():
        pltpu.sync_copy(hist_sp, hist.at[c])


def _k0b_body(h_ref, dis_ref):
    h = h_ref[...]
    deg = h[0:1, :] + h[1:2, :] + 1.0
    dis_ref[...] = lax.rsqrt(deg)


_k0b = pl.pallas_call(
    _k0b_body,
    out_shape=jax.ShapeDtypeStruct((1, NP), _f32),
)


def _k1a_body(h_ref, w_ref, b_ref, o_ref):
    x = h_ref[...]
    o_ref[...] = (
        jnp.dot(x, w_ref[...], preferred_element_type=_f32) + b_ref[...]
    )


def _k1b_body(h_ref, st_ref, w_ref, b_ref, o_ref):
    x = h_ref[...]
    x = jnp.maximum(x * st_ref[0:1, :] + st_ref[1:2, :], 0.0)
    o_ref[...] = (
        jnp.dot(x, w_ref[...], preferred_element_type=_f32) + b_ref[...]
    )


_k1a = pl.pallas_call(
    _k1a_body,
    grid=(NP // BM,),
    in_specs=[
        pl.BlockSpec((BM, D), lambda i: (i, 0)),
        pl.BlockSpec((D, D), lambda i: (0, 0)),
        pl.BlockSpec((1, D), lambda i: (0, 0)),
    ],
    out_specs=pl.BlockSpec((BM, D), lambda i: (i, 0)),
    out_shape=jax.ShapeDtypeStruct((NP, D), _f32),
)

_k1b = pl.pallas_call(
    _k1b_body,
    grid=(NP // BM,),
    in_specs=[
        pl.BlockSpec((BM, D), lambda i: (i, 0)),
        pl.BlockSpec((2, D), lambda i: (0, 0)),
        pl.BlockSpec((D, D), lambda i: (0, 0)),
        pl.BlockSpec((1, D), lambda i: (0, 0)),
    ],
    out_specs=pl.BlockSpec((BM, D), lambda i: (i, 0)),
    out_shape=jax.ShapeDtypeStruct((NP, D), _f32),
)


@functools.partial(
    pl.kernel,
    mesh=_MESH,
    compiler_params=_SC_PARAMS,
    out_type=jax.ShapeDtypeStruct((NP, D), _f32),
    scratch_types=[
        pltpu.VMEM((NB, D), _f32),
        pltpu.VMEM((NP + 16,), _f32),
        pltpu.VMEM((3, D), _f32),
        pltpu.VMEM((KC + 16,), _i32),
        pltpu.VMEM((KC + 16,), _i32),
        pltpu.VMEM((KC + 16,), _f32),
        pltpu.VMEM((KC + 16,), _f32),
        pltpu.VMEM((KC, D), _f32),
        pltpu.VMEM((16,), _i32),
        pltpu.SemaphoreType.DMA,
    ],
)
def _k2(xl, rowr, colr, a0r, a1r, tbr, disr, wr, outp, acc, dis_v, w_v,
        rowi_v, coli_v, a0_v, a1_v, rows_v, tb_v, sem):
    c = lax.axis_index("c")
    s = lax.axis_index("s")
    w = c * 16 + s
    base = w * NB

    pltpu.sync_copy(disr, dis_v.at[pl.ds(0, NP)])
    pltpu.sync_copy(wr, w_v)
    pltpu.sync_copy(tbr.at[pl.ds(pl.multiple_of(w * 16, 8), 16)], tb_v)

    lane = lax.iota(_i32, 16)
    zf = jnp.zeros((16,), _f32)
    tbv = tb_v[...]
    lo_al = jnp.sum(jnp.where(lane == 0, tbv, 0))
    lo = jnp.sum(jnp.where(lane == 1, tbv, 0))
    hi = jnp.sum(jnp.where(lane == 2, tbv, 0))

    w0s = [w_v[0, pl.ds(k * 16, 16)] for k in range(D // 16)]
    w1s = [w_v[1, pl.ds(k * 16, 16)] for k in range(D // 16)]

    def _zrow(r, _):
        for k in range(D // 16):
            acc[r, pl.ds(k * 16, 16)] = zf
        return 0

    lax.fori_loop(0, NB, _zrow, 0)

    nch = (hi - lo_al + KC - 1) >> 7

    def _flush(ccol, accs):
        cl = jnp.minimum(jnp.maximum(ccol - base, 0), NB - 1)
        for k in range(D // 16):
            plsc.addupdate(acc.at[cl, pl.ds(k * 16, 16)], accs[k])

    def _chunk(ci, carry):
        e0 = pl.multiple_of(lo_al + ci * KC, 8)
        pltpu.async_copy(rowr.at[pl.ds(e0, KC)], rowi_v.at[pl.ds(0, KC)],
                         sem).wait()
        cps = [
            pltpu.async_copy(colr.at[pl.ds(e0, KC)],
                             coli_v.at[pl.ds(0, KC)], sem),
            pltpu.async_copy(a0r.at[pl.ds(e0, KC)],
                             a0_v.at[pl.ds(0, KC)], sem),
            pltpu.async_copy(a1r.at[pl.ds(e0, KC)],
                             a1_v.at[pl.ds(0, KC)], sem),
            pltpu.async_copy(xl.at[rowi_v.at[pl.ds(0, KC)]], rows_v, sem),
        ]
        for cp in cps:
            cp.wait()

        def _estep(j, ccol, accs):
            ncol = coli_v[pl.ds(j, 16)][0]
            change = ncol != ccol

            def _do_flush():
                _flush(ccol, accs)
                return [zf] * (D // 16)

            accs = lax.cond(change, _do_flush, lambda: accs)

            eid = e0 + j
            valid = (eid >= lo) & (eid < hi)
            r = rowi_v[pl.ds(j, 16)][0]
            dr = dis_v[pl.ds(r, 16)][0]
            dc = dis_v[pl.ds(ncol, 16)][0]
            nrm = jnp.where(valid, dr * dc, 0.0)
            nrmb = jnp.full((16,), nrm)
            a0b = jnp.full((16,), a0_v[pl.ds(j, 16)][0])
            a1b = jnp.full((16,), a1_v[pl.ds(j, 16)][0])
            nacc = []
            for k in range(D // 16):
                feat = rows_v[j, pl.ds(k * 16, 16)]
                m = nrmb * jnp.maximum(feat + a0b * w0s[k] + a1b * w1s[k],
                                       0.0)
                nacc.append(accs[k] + m)
            return ncol, nacc

        def _edge(jj, carry):
            ccol, accs = carry[0], list(carry[1:])
            for u in range(4):
                ccol, accs = _estep(4 * jj + u, ccol, accs)
            return tuple([ccol] + accs)

        return lax.fori_loop(0, KC // 4, _edge, carry)

    init = tuple([base] + [zf] * (D // 16))
    fin = lax.fori_loop(0, nch, _chunk, init)
    _flush(fin[0], list(fin[1:]))

    rts = [w_v[2, pl.ds(k * 16, 16)] for k in range(D // 16)]
    for oc in range(NB // 80):
        o = pl.multiple_of(base + oc * 80, 8)
        pltpu.sync_copy(xl.at[pl.ds(o, 80)], rows_v.at[pl.ds(0, 80)])

        def _snode(j, _):
            nid = o + j
            dl = dis_v[pl.ds(nid, 16)][0]
            invb = jnp.full((16,), jnp.where(nid < N, dl * dl, 0.0))
            for k in range(D // 16):
                feat = rows_v[j, pl.ds(k * 16, 16)]
                sv = invb * jnp.maximum(feat + rts[k], 0.0)
                plsc.addupdate(acc.at[oc * 80 + j, pl.ds(k * 16, 16)], sv)
            return 0

        lax.fori_loop(0, 80, _snode, 0)

    pltpu.sync_copy(acc, outp.at[pl.ds(pl.multiple_of(base, 8), NB)])


def _k3_body(o_ref, g_ref, b_ref, st_ref, acc_ref):
    i = pl.program_id(0)

    @pl.when(i == 0)
    def _():
        acc_ref[...] = jnp.zeros((2, D), _f32)

    rid = lax.broadcasted_iota(_i32, (BM, 1), 0) + i * BM
    x = jnp.where(rid < N, o_ref[...], 0.0)
    acc_ref[0:1, :] += jnp.sum(x, axis=0, keepdims=True)
    acc_ref[1:2, :] += jnp.sum(x * x, axis=0, keepdims=True)

    @pl.when(i == NP // BM - 1)
    def _():
        mean = acc_ref[0:1, :] * (1.0 / N)
        var = acc_ref[1:2, :] * (1.0 / N) - mean * mean
        sca = g_ref[...] * lax.rsqrt(var + 1e-5)
        st_ref[...] = jnp.concatenate([sca, b_ref[...] - mean * sca], axis=0)


_k3 = pl.pallas_call(
    _k3_body,
    grid=(NP // BM,),
    in_specs=[
        pl.BlockSpec((BM, D), lambda i: (i, 0)),
        pl.BlockSpec((1, D), lambda i: (0, 0)),
        pl.BlockSpec((1, D), lambda i: (0, 0)),
    ],
    out_specs=pl.BlockSpec((2, D), lambda i: (0, 0)),
    out_shape=jax.ShapeDtypeStruct((2, D), _f32),
    scratch_shapes=[pltpu.VMEM((2, D), _f32)],
)


def _k4_body(o_ref, st_ref, h_ref):
    h_ref[...] = o_ref[...] * st_ref[0:1, :] + st_ref[1:2, :]


_k4 = pl.pallas_call(
    _k4_body,
    grid=(10,),
    in_specs=[
        pl.BlockSpec((N // 10, D), lambda i: (i, 0)),
        pl.BlockSpec((2, D), lambda i: (0, 0)),
    ],
    out_specs=pl.BlockSpec((N // 10, D), lambda i: (i, 0)),
    out_shape=jax.ShapeDtypeStruct((N, D), _f32),
)


def kernel(x, edge_index, edge_attr, node_depth, batch, node_table,
           depth_table, Ws, bs, roots, Wes, bes, gammas, betas):
    row = edge_index[0].astype(_i32)
    col = edge_index[1].astype(_i32)

    col_s, order = lax.sort_key_val(col, lax.iota(_i32, E))
    row_s = row[order]
    a0_s = edge_attr[order, 0]
    a1_s = edge_attr[order, 1]
    pad = EP - E
    row_p = jnp.concatenate([row_s, jnp.zeros((pad,), _i32)])
    col_p = jnp.concatenate([col_s, jnp.zeros((pad,), _i32)])
    a0_p = jnp.concatenate([a0_s, jnp.zeros((pad,), _f32)])
    a1_p = jnp.concatenate([a1_s, jnp.zeros((pad,), _f32)])

    bounds = jnp.searchsorted(col_s, jnp.arange(NW + 1, dtype=_i32) * NB)
    bounds = bounds.astype(_i32)
    lo = bounds[:NW]
    hi = bounds[1:]
    tb = jnp.stack(
        [(lo >> 3) << 3, lo, hi] + [jnp.zeros((NW,), _i32)] * 13, axis=1
    ).reshape(-1)

    x_pad = jnp.concatenate([x.astype(_i32), jnp.zeros((NP - N,), _i32)])
    nd_pad = jnp.concatenate(
        [node_depth.reshape(-1).astype(_i32), jnp.zeros((NP - N,), _i32)]
    )

    h0, hist = _k0a(x_pad, nd_pad, node_table, depth_table, row_p)
    dis = _k0b(hist).reshape(NP)

    outp = h0
    st = None
    for l in range(L):
        wbuf = jnp.stack([Wes[l, 0], Wes[l, 1], roots[l] - bes[l]], axis=0)
        beff = (bs[l] + bes[l]).reshape(1, D)
        if l == 0:
            xl = _k1a(h0, Ws[0], beff)
        else:
            xl = _k1b(outp, st, Ws[l], beff)
        outp = _k2(xl, row_p, col_p, a0_p, a1_p, tb, dis, wbuf)
        st = _k3(outp, gammas[l].reshape(1, D), betas[l].reshape(1, D))
    return _k4(outp, st)


# speedup vs baseline: 1.1267x; 1.1267x over previous
import functools

import jax
import jax.numpy as jnp
from jax import lax
from jax.experimental import pallas as pl
from jax.experimental.pallas import tpu as pltpu
from jax.experimental.pallas import tpu_sc as plsc

N = 10000
E = 160000
D = 256
L = 5

NW = 32
NB = 320
NP = NW * NB
KC = 128
EP = E + 2 * KC
ET = E // NW
KH = 200
KE = 64
BM = 1024

_SC_PARAMS = pltpu.CompilerParams(
    use_tc_tiling_on_sc=False, needs_layout_passes=False
)
_MESH = plsc.VectorSubcoreMesh(core_axis_name="c", subcore_axis_name="s")

_f32 = jnp.float32
_i32 = jnp.int32


@functools.partial(
    pl.kernel,
    mesh=_MESH,
    compiler_params=_SC_PARAMS,
    out_type=(
        jax.ShapeDtypeStruct((NP, D), _f32),
        jax.ShapeDtypeStruct((2, NP), _f32),
    ),
    scratch_types=[
        pltpu.VMEM((KE,), _i32),
        pltpu.VMEM((KE,), _i32),
        pltpu.VMEM((KE, D), _f32),
        pltpu.VMEM((KE, D), _f32),
        pltpu.VMEM((KH,), _i32),
        pltpu.VMEM((208,), _f32),
        pltpu.VMEM((NP // 16,), _f32),
        pltpu.VMEM_SHARED((NP,), _f32),
        pltpu.SemaphoreType.DMA,
    ],
)
def _k0a(xi, ndi, ntab, dtab, rowr, h0, hist, idx_v, idx2_v, t1_v, t2_v,
         hidx_v, ones_v, zb_v, hist_sp, sem):
    c = lax.axis_index("c")
    s = lax.axis_index("s")
    w = c * 16 + s
    base = w * NB
    zslice = NP // 16

    zf = jnp.zeros((16,), _f32)
    of = zf + 1.0

    def _zb(i, _):
        zb_v[pl.ds(i * 16, 16)] = zf
        return 0

    lax.fori_loop(0, zslice // 16, _zb, 0)
    for k in range(13):
        ones_v[pl.ds(k * 16, 16)] = of

    pltpu.sync_copy(zb_v, hist_sp.at[pl.ds(pl.multiple_of(s * zslice, 8), zslice)])
    plsc.subcore_barrier()

    ebase = w * ET

    def _hchunk(ci, _):
        e0 = pl.multiple_of(ebase + ci * KH, 8)
        pltpu.sync_copy(rowr.at[pl.ds(e0, KH)], hidx_v)
        pltpu.sync_copy(ones_v.at[pl.ds(0, KH)], hist_sp.at[hidx_v],
                        add=True)
        return 0

    lax.fori_loop(0, ET // KH, _hchunk, 0)

    for oc in range(NB // KE):
        o = pl.multiple_of(base + oc * KE, 8)
        c1 = pltpu.async_copy(xi.at[pl.ds(o, KE)], idx_v, sem)
        c2 = pltpu.async_copy(ndi.at[pl.ds(o, KE)], idx2_v, sem)
        c1.wait()
        c2.wait()
        c3 = pltpu.async_copy(ntab.at[idx_v], t1_v, sem)
        c4 = pltpu.async_copy(dtab.at[idx2_v], t2_v, sem)
        c3.wait()
        c4.wait()

        def _addrow(r, _):
            for k in range(D // 16):
                t1_v[r, pl.ds(k * 16, 16)] = (
                    t1_v[r, pl.ds(k * 16, 16)] + t2_v[r, pl.ds(k * 16, 16)]
                )
            return 0

        lax.fori_loop(0, KE, _addrow, 0)
        pltpu.sync_copy(t1_v, h0.at[pl.ds(o, KE)])

    plsc.subcore_barrier()

    @pl.when(s == 0)
    def _():
        pltpu.sync_copy(hist_sp, hist.at[c])


def _k0b_body(h_ref, dis_ref):
    h = h_ref[...]
    deg = h[0:1, :] + h[1:2, :] + 1.0
    dis_ref[...] = lax.rsqrt(deg)


_k0b = pl.pallas_call(
    _k0b_body,
    out_shape=jax.ShapeDtypeStruct((1, NP), _f32),
)


def _k1a_body(h_ref, w_ref, b_ref, o_ref):
    x = h_ref[...]
    o_ref[...] = (
        jnp.dot(x, w_ref[...], preferred_element_type=_f32) + b_ref[...]
    )


def _k1b_body(h_ref, st_ref, w_ref, b_ref, o_ref):
    x = h_ref[...]
    x = jnp.maximum(x * st_ref[0:1, :] + st_ref[1:2, :], 0.0)
    o_ref[...] = (
        jnp.dot(x, w_ref[...], preferred_element_type=_f32) + b_ref[...]
    )


_k1a = pl.pallas_call(
    _k1a_body,
    grid=(NP // BM,),
    in_specs=[
        pl.BlockSpec((BM, D), lambda i: (i, 0)),
        pl.BlockSpec((D, D), lambda i: (0, 0)),
        pl.BlockSpec((1, D), lambda i: (0, 0)),
    ],
    out_specs=pl.BlockSpec((BM, D), lambda i: (i, 0)),
    out_shape=jax.ShapeDtypeStruct((NP, D), _f32),
)

_k1b = pl.pallas_call(
    _k1b_body,
    grid=(NP // BM,),
    in_specs=[
        pl.BlockSpec((BM, D), lambda i: (i, 0)),
        pl.BlockSpec((2, D), lambda i: (0, 0)),
        pl.BlockSpec((D, D), lambda i: (0, 0)),
        pl.BlockSpec((1, D), lambda i: (0, 0)),
    ],
    out_specs=pl.BlockSpec((BM, D), lambda i: (i, 0)),
    out_shape=jax.ShapeDtypeStruct((NP, D), _f32),
)


@functools.partial(
    pl.kernel,
    mesh=_MESH,
    compiler_params=_SC_PARAMS,
    out_type=jax.ShapeDtypeStruct((NP, D), _f32),
    scratch_types=[
        pltpu.VMEM((NB, D), _f32),
        pltpu.VMEM((NP + 16,), _f32),
        pltpu.VMEM((3, D), _f32),
        pltpu.VMEM((KC + 16,), _i32),
        pltpu.VMEM((KC + 16,), _i32),
        pltpu.VMEM((KC + 16,), _f32),
        pltpu.VMEM((KC + 16,), _f32),
        pltpu.VMEM((KC, D), _f32),
        pltpu.VMEM((16,), _i32),
        pltpu.SemaphoreType.DMA,
    ],
)
def _k2(xl, rowr, colr, a0r, a1r, tbr, disr, wr, outp, acc, dis_v, w_v,
        rowi_v, coli_v, a0_v, a1_v, rows_v, tb_v, sem):
    c = lax.axis_index("c")
    s = lax.axis_index("s")
    w = c * 16 + s
    base = w * NB

    pltpu.sync_copy(disr, dis_v.at[pl.ds(0, NP)])
    pltpu.sync_copy(wr, w_v)
    pltpu.sync_copy(tbr.at[pl.ds(pl.multiple_of(w * 16, 8), 16)], tb_v)

    lane = lax.iota(_i32, 16)
    zf = jnp.zeros((16,), _f32)
    tbv = tb_v[...]
    lo_al = jnp.sum(jnp.where(lane == 0, tbv, 0))
    lo = jnp.sum(jnp.where(lane == 1, tbv, 0))
    hi = jnp.sum(jnp.where(lane == 2, tbv, 0))

    w0s = [w_v[0, pl.ds(k * 16, 16)] for k in range(D // 16)]
    w1s = [w_v[1, pl.ds(k * 16, 16)] for k in range(D // 16)]

    def _zrow(r, _):
        for k in range(D // 16):
            acc[r, pl.ds(k * 16, 16)] = zf
        return 0

    lax.fori_loop(0, NB, _zrow, 0)

    nch = (hi - lo_al + KC - 1) >> 7

    def _flush(ccol, accs):
        cl = jnp.minimum(jnp.maximum(ccol - base, 0), NB - 1)
        for k in range(D // 16):
            plsc.addupdate(acc.at[cl, pl.ds(k * 16, 16)], accs[k])

    def _chunk(ci, carry):
        e0 = pl.multiple_of(lo_al + ci * KC, 8)
        pltpu.async_copy(rowr.at[pl.ds(e0, KC)], rowi_v.at[pl.ds(0, KC)],
                         sem).wait()
        cps = [
            pltpu.async_copy(colr.at[pl.ds(e0, KC)],
                             coli_v.at[pl.ds(0, KC)], sem),
            pltpu.async_copy(a0r.at[pl.ds(e0, KC)],
                             a0_v.at[pl.ds(0, KC)], sem),
            pltpu.async_copy(a1r.at[pl.ds(e0, KC)],
                             a1_v.at[pl.ds(0, KC)], sem),
            pltpu.async_copy(xl.at[rowi_v.at[pl.ds(0, KC)]], rows_v, sem),
        ]
        for cp in cps:
            cp.wait()

        def _estep(j, ccol, accs):
            ncol = coli_v[pl.ds(j, 16)][0]
            change = ncol != ccol

            def _do_flush():
                _flush(ccol, accs)
                return [zf] * (D // 16)

            accs = lax.cond(change, _do_flush, lambda: accs)

            eid = e0 + j
            valid = (eid >= lo) & (eid < hi)
            r = rowi_v[pl.ds(j, 16)][0]
            dr = dis_v[pl.ds(r, 16)][0]
            dc = dis_v[pl.ds(ncol, 16)][0]
            nrm = jnp.where(valid, dr * dc, 0.0)
            nrmb = jnp.full((16,), nrm)
            a0b = jnp.full((16,), a0_v[pl.ds(j, 16)][0])
            a1b = jnp.full((16,), a1_v[pl.ds(j, 16)][0])
            nacc = []
            for k in range(D // 16):
                feat = rows_v[j, pl.ds(k * 16, 16)]
                m = nrmb * jnp.maximum(feat + a0b * w0s[k] + a1b * w1s[k],
                                       0.0)
                nacc.append(accs[k] + m)
            return ncol, nacc

        def _edge(jj, carry):
            ccol, accs = carry[0], list(carry[1:])
            ccol, accs = _estep(2 * jj, ccol, accs)
            ccol, accs = _estep(2 * jj + 1, ccol, accs)
            return tuple([ccol] + accs)

        return lax.fori_loop(0, KC // 2, _edge, carry)

    init = tuple([base] + [zf] * (D // 16))
    fin = lax.fori_loop(0, nch, _chunk, init)
    _flush(fin[0], list(fin[1:]))

    rts = [w_v[2, pl.ds(k * 16, 16)] for k in range(D // 16)]
    for oc in range(NB // 80):
        o = pl.multiple_of(base + oc * 80, 8)
        pltpu.sync_copy(xl.at[pl.ds(o, 80)], rows_v.at[pl.ds(0, 80)])

        def _snode(j, _):
            nid = o + j
            dl = dis_v[pl.ds(nid, 16)][0]
            invb = jnp.full((16,), jnp.where(nid < N, dl * dl, 0.0))
            for k in range(D // 16):
                feat = rows_v[j, pl.ds(k * 16, 16)]
                sv = invb * jnp.maximum(feat + rts[k], 0.0)
                plsc.addupdate(acc.at[oc * 80 + j, pl.ds(k * 16, 16)], sv)
            return 0

        lax.fori_loop(0, 80, _snode, 0)

    pltpu.sync_copy(acc, outp.at[pl.ds(pl.multiple_of(base, 8), NB)])


def _k3_body(o_ref, g_ref, b_ref, st_ref, acc_ref):
    i = pl.program_id(0)

    @pl.when(i == 0)
    def _():
        acc_ref[...] = jnp.zeros((2, D), _f32)

    rid = lax.broadcasted_iota(_i32, (BM, 1), 0) + i * BM
    x = jnp.where(rid < N, o_ref[...], 0.0)
    acc_ref[0:1, :] += jnp.sum(x, axis=0, keepdims=True)
    acc_ref[1:2, :] += jnp.sum(x * x, axis=0, keepdims=True)

    @pl.when(i == NP // BM - 1)
    def _():
        mean = acc_ref[0:1, :] * (1.0 / N)
        var = acc_ref[1:2, :] * (1.0 / N) - mean * mean
        sca = g_ref[...] * lax.rsqrt(var + 1e-5)
        st_ref[...] = jnp.concatenate([sca, b_ref[...] - mean * sca], axis=0)


_k3 = pl.pallas_call(
    _k3_body,
    grid=(NP // BM,),
    in_specs=[
        pl.BlockSpec((BM, D), lambda i: (i, 0)),
        pl.BlockSpec((1, D), lambda i: (0, 0)),
        pl.BlockSpec((1, D), lambda i: (0, 0)),
    ],
    out_specs=pl.BlockSpec((2, D), lambda i: (0, 0)),
    out_shape=jax.ShapeDtypeStruct((2, D), _f32),
    scratch_shapes=[pltpu.VMEM((2, D), _f32)],
)


def _k4_body(o_ref, st_ref, h_ref):
    h_ref[...] = o_ref[...] * st_ref[0:1, :] + st_ref[1:2, :]


_k4 = pl.pallas_call(
    _k4_body,
    grid=(10,),
    in_specs=[
        pl.BlockSpec((N // 10, D), lambda i: (i, 0)),
        pl.BlockSpec((2, D), lambda i: (0, 0)),
    ],
    out_specs=pl.BlockSpec((N // 10, D), lambda i: (i, 0)),
    out_shape=jax.ShapeDtypeStruct((N, D), _f32),
)


def kernel(x, edge_index, edge_attr, node_depth, batch, node_table,
           depth_table, Ws, bs, roots, Wes, bes, gammas, betas):
    row = edge_index[0].astype(_i32)
    col = edge_index[1].astype(_i32)

    col_s, order = lax.sort_key_val(col, lax.iota(_i32, E))
    row_s = row[order]
    a0_s = edge_attr[order, 0]
    a1_s = edge_attr[order, 1]
    pad = EP - E
    row_p = jnp.concatenate([row_s, jnp.zeros((pad,), _i32)])
    col_p = jnp.concatenate([col_s, jnp.zeros((pad,), _i32)])
    a0_p = jnp.concatenate([a0_s, jnp.zeros((pad,), _f32)])
    a1_p = jnp.concatenate([a1_s, jnp.zeros((pad,), _f32)])

    bounds = jnp.searchsorted(col_s, jnp.arange(NW + 1, dtype=_i32) * NB)
    bounds = bounds.astype(_i32)
    lo = bounds[:NW]
    hi = bounds[1:]
    tb = jnp.stack(
        [(lo >> 3) << 3, lo, hi] + [jnp.zeros((NW,), _i32)] * 13, axis=1
    ).reshape(-1)

    x_pad = jnp.concatenate([x.astype(_i32), jnp.zeros((NP - N,), _i32)])
    nd_pad = jnp.concatenate(
        [node_depth.reshape(-1).astype(_i32), jnp.zeros((NP - N,), _i32)]
    )

    h0, hist = _k0a(x_pad, nd_pad, node_table, depth_table, row_p)
    dis = _k0b(hist).reshape(NP)

    outp = h0
    st = None
    for l in range(L):
        wbuf = jnp.stack([Wes[l, 0], Wes[l, 1], roots[l] - bes[l]], axis=0)
        beff = (bs[l] + bes[l]).reshape(1, D)
        if l == 0:
            xl = _k1a(h0, Ws[0], beff)
        else:
            xl = _k1b(outp, st, Ws[l], beff)
        outp = _k2(xl, row_p, col_p, a0_p, a1_p, tb, dis, wbuf)
        st = _k3(outp, gammas[l].reshape(1, D), betas[l].reshape(1, D))
    return _k4(outp, st)
